# baseline (speedup 1.0000x reference)
import functools

import jax
import jax.numpy as jnp
from jax import lax
from jax.experimental import pallas as pl
from jax.experimental.pallas import tpu as pltpu
from jax.experimental.pallas import tpu_sc as plsc

_NC = 2
_NS = 16
_L = 16
_NW = _NC * _NS


def kernel(target, vector_embedding, centers):
    B, D = vector_embedding.shape
    V = centers.shape[0]
    n_pass = D // _NW
    BH = B // 2

    cen_t = centers.T
    emb_t = vector_embedding.T

    mesh = plsc.VectorSubcoreMesh(core_axis_name="c", subcore_axis_name="s")

    @functools.partial(
        pl.kernel,
        mesh=mesh,
        out_type=jax.ShapeDtypeStruct((_NW, _L), jnp.float32),
        scratch_types=[
            pltpu.VMEM((V,), jnp.float32),
            pltpu.VMEM((BH,), jnp.float32),
            pltpu.VMEM((B,), jnp.int32),
            pltpu.VMEM((_L,), jnp.float32),
            pltpu.SemaphoreType.DMA,
        ],
        compiler_params=pltpu.CompilerParams(needs_layout_passes=False),
    )
    def sc_kernel(tgt_hbm, emb_hbm, cen_hbm, out_hbm, row_v, emb_v, idx_v,
                  acc_v, sem):
        wid = lax.axis_index("s") * _NC + lax.axis_index("c")

        pltpu.sync_copy(tgt_hbm, idx_v)
        zero = jnp.zeros((_L,), jnp.float32)
        accs = (zero, zero)
        for p in range(n_pass):
            f = wid + p * _NW
            pltpu.sync_copy(cen_hbm.at[f], row_v)
            for h in range(2):
                pltpu.sync_copy(emb_hbm.at[f, pl.ds(h * BH, BH)], emb_v)
                ib = h * BH

                @plsc.parallel_loop(0, BH // (2 * _L), unroll=8, carry=accs)
                def body(i, accs):
                    a0, a1 = accs
                    b = i * (2 * _L)
                    t0 = idx_v[pl.ds(ib + b, _L)]
                    e0 = emb_v[pl.ds(b, _L)]
                    g0 = plsc.load_gather(row_v, [t0])
                    d0 = e0 - g0
                    t1 = idx_v[pl.ds(ib + b + _L, _L)]
                    e1 = emb_v[pl.ds(b + _L, _L)]
                    g1 = plsc.load_gather(row_v, [t1])
                    d1 = e1 - g1
                    return (a0 + d0 * d0, a1 + d1 * d1)

                accs = body

        acc_v[...] = (accs[0] + accs[1]) * (0.5 / B)
        pltpu.sync_copy(acc_v, out_hbm.at[wid])

    partials = sc_kernel(target, emb_t, cen_t)
    return jnp.sum(partials)


# speedup vs baseline: 1.0863x; 1.0863x over previous
import functools

import jax
import jax.numpy as jnp
from jax import lax
from jax.experimental import pallas as pl
from jax.experimental.pallas import tpu as pltpu
from jax.experimental.pallas import tpu_sc as plsc

_NC = 2
_NS = 16
_L = 16
_NW = _NC * _NS


def kernel(target, vector_embedding, centers):
    B, D = vector_embedding.shape
    V = centers.shape[0]
    n_pass = D // _NW
    n_q = 4
    BQ = B // n_q

    cen_t = centers.T
    emb_t = vector_embedding.T

    mesh = plsc.VectorSubcoreMesh(core_axis_name="c", subcore_axis_name="s")

    @functools.partial(
        pl.kernel,
        mesh=mesh,
        out_type=jax.ShapeDtypeStruct((_NW, _L), jnp.float32),
        scratch_types=[
            pltpu.VMEM((V,), jnp.float32),
            pltpu.VMEM((2, BQ), jnp.float32),
            pltpu.VMEM((B,), jnp.int32),
            pltpu.VMEM((_L,), jnp.float32),
            pltpu.SemaphoreType.DMA,
            pltpu.SemaphoreType.DMA,
            pltpu.SemaphoreType.DMA,
            pltpu.SemaphoreType.DMA,
        ],
        compiler_params=pltpu.CompilerParams(needs_layout_passes=False),
    )
    def sc_kernel(tgt_hbm, emb_hbm, cen_hbm, out_hbm, row_v, emb_v, idx_v,
                  acc_v, sem_i, sem_r, sem_e0, sem_e1):
        wid = lax.axis_index("s") * _NC + lax.axis_index("c")
        esems = (sem_e0, sem_e1)

        def emb_cp(p, q):
            gq = p * n_q + q
            return pltpu.async_copy(
                emb_hbm.at[wid + p * _NW, pl.ds(q * BQ, BQ)],
                emb_v.at[gq % 2], esems[gq % 2])

        cp_i = pltpu.async_copy(tgt_hbm, idx_v, sem_i)
        cp_r = pltpu.async_copy(cen_hbm.at[wid], row_v, sem_r)
        pending = [emb_cp(0, 0), emb_cp(0, 1)]
        cp_i.wait()
        cp_r.wait()

        zero = jnp.zeros((_L,), jnp.float32)
        accs = (zero, zero)
        for p in range(n_pass):
            for q in range(n_q):
                gq = p * n_q + q
                pending[gq % 2].wait()
                buf = gq % 2
                ib = q * BQ

                @plsc.parallel_loop(0, BQ // (2 * _L), unroll=4, carry=accs)
                def body(i, accs):
                    a0, a1 = accs
                    b = i * (2 * _L)
                    t0 = idx_v[pl.ds(ib + b, _L)]
                    e0 = emb_v[buf, pl.ds(b, _L)]
                    g0 = plsc.load_gather(row_v, [t0])
                    d0 = e0 - g0
                    t1 = idx_v[pl.ds(ib + b + _L, _L)]
                    e1 = emb_v[buf, pl.ds(b + _L, _L)]
                    g1 = plsc.load_gather(row_v, [t1])
                    d1 = e1 - g1
                    return (a0 + d0 * d0, a1 + d1 * d1)

                accs = body
                ngq = gq + 2
                if ngq < n_pass * n_q:
                    pending[gq % 2] = emb_cp(ngq // n_q, ngq % n_q)
            if p + 1 < n_pass:
                pltpu.sync_copy(cen_hbm.at[wid + (p + 1) * _NW], row_v)

        acc_v[...] = (accs[0] + accs[1]) * (0.5 / B)
        pltpu.sync_copy(acc_v, out_hbm.at[wid])

    partials = sc_kernel(target, emb_t, cen_t)
    return jnp.sum(partials)
